# Initial kernel scaffold; baseline (speedup 1.0000x reference)
import jax
import jax.numpy as jnp
from jax.experimental import pallas as pl


def kernel(inputs, idx, fc11_w, fc11_b, fc12_w, fc12_b, fc21_w, fc21_b, mean_bn_w, mean_bn_b, fcd1_w, dec_bn_w, dec_bn_b, codebook, theta_bank, M_cos_dist, M_coo_dist, training_data, is_aug):
    raise NotImplementedError("write your pallas kernel here")


# speedup vs baseline: 6.6611x; 6.6611x over previous
import functools

import jax
import jax.numpy as jnp
from jax import lax
from jax.experimental import pallas as pl
from jax.experimental.pallas import tpu as pltpu
from jax.experimental.pallas import tpu_sc as plsc

B = 1024
V = 10000
N = 4096
T = 50
EU = 200
K = 20
ALPHA = 1.0
ETA = 0.5
RHO = 0.5
EPS = 1e-5

_F32 = jnp.float32


_CH = 8
_N_CH = 4

_sc_mesh = plsc.VectorSubcoreMesh(core_axis_name="c", subcore_axis_name="s")


@functools.partial(
    pl.kernel,
    mesh=_sc_mesh,
    out_type=(
        jax.ShapeDtypeStruct((B, N), _F32),
        jax.ShapeDtypeStruct((B, N), _F32),
    ),
    scratch_types=[
        pltpu.VMEM((_CH,), jnp.int32),
        pltpu.VMEM((_CH, N), _F32),
        pltpu.VMEM((_CH, N), _F32),
        pltpu.SemaphoreType.DMA,
        pltpu.SemaphoreType.DMA,
    ],
)
def _bow_gather(idx_hbm, cos_hbm, coo_hbm, cosg_hbm, coog_hbm,
                idx_v, a_v, b_v, sem_a, sem_b):
    wid = lax.axis_index("s") * 2 + lax.axis_index("c")
    base = wid * (B // 32)

    def chunk(c, carry):
        off = pl.multiple_of(base + c * _CH, _CH)
        pltpu.sync_copy(idx_hbm.at[pl.ds(off, _CH)], idx_v)
        cp_a = pltpu.async_copy(cos_hbm.at[idx_v], a_v, sem_a)
        cp_b = pltpu.async_copy(coo_hbm.at[idx_v], b_v, sem_b)
        cp_a.wait()
        cp_b.wait()
        pltpu.sync_copy(a_v, cosg_hbm.at[pl.ds(off, _CH)])
        pltpu.sync_copy(b_v, coog_hbm.at[pl.ds(off, _CH)])
        return carry

    lax.fori_loop(0, _N_CH, chunk, 0)


def _k1_body(x_ref, w_ref, b_ref, e1_ref, xsum_ref):
    x = x_ref[...]
    acc = lax.dot_general(x, w_ref[...], (((1,), (1,)), ((), ())),
                          preferred_element_type=_F32)
    e1_ref[...] = jnp.logaddexp(acc + b_ref[...], 0.0)
    xsum_ref[...] = x.sum(axis=1, keepdims=True)


def _k1(inputs, fc11_w, fc11_b):
    bm = 256
    return pl.pallas_call(
        _k1_body,
        grid=(B // bm,),
        in_specs=[
            pl.BlockSpec((bm, V), lambda i: (i, 0)),
            pl.BlockSpec((EU, V), lambda i: (0, 0)),
            pl.BlockSpec((1, EU), lambda i: (0, 0)),
        ],
        out_specs=[
            pl.BlockSpec((bm, EU), lambda i: (i, 0)),
            pl.BlockSpec((bm, 1), lambda i: (i, 0)),
        ],
        out_shape=[
            jax.ShapeDtypeStruct((B, EU), _F32),
            jax.ShapeDtypeStruct((B, 1), _F32),
        ],
    )(inputs, fc11_w, fc11_b)


def _k2_body(e1_ref, w12_ref, b12_ref, w21_ref, b21_ref, bnw_ref, bnb_ref,
             cb_ref, st_ref, oh_ref, cnt_ref, vq_ref):
    e1 = e1_ref[...]
    e2 = jnp.logaddexp(
        lax.dot_general(e1, w12_ref[...], (((1,), (1,)), ((), ())),
                        preferred_element_type=_F32) + b12_ref[...], 0.0)
    h = lax.dot_general(e2, w21_ref[...], (((1,), (1,)), ((), ())),
                        preferred_element_type=_F32) + b21_ref[...]
    mu = jnp.mean(h, axis=0, keepdims=True)
    var = jnp.mean((h - mu) ** 2, axis=0, keepdims=True)
    theta = (h - mu) / jnp.sqrt(var + EPS) * bnw_ref[...] + bnb_ref[...]
    tmax = theta.max(axis=1, keepdims=True)
    ex = jnp.exp(theta - tmax)
    st = ex / ex.sum(axis=1, keepdims=True)
    st_ref[...] = st
    cb = cb_ref[...]
    d = ((st * st).sum(axis=1, keepdims=True)
         + (cb * cb).sum(axis=1, keepdims=True).reshape(1, T)
         - 2.0 * lax.dot_general(st, cb, (((1,), (1,)), ((), ())),
                                 preferred_element_type=_F32))
    dmin = d.min(axis=1, keepdims=True)
    ii = lax.broadcasted_iota(jnp.int32, d.shape, 1)
    a = jnp.where(d == dmin, ii, T).min(axis=1, keepdims=True)
    oh = (ii == a).astype(_F32)
    oh_ref[...] = oh
    cnt_ref[...] = oh.sum(axis=0, keepdims=True)
    q = lax.dot_general(oh, cb, (((1,), (0,)), ((), ())),
                        preferred_element_type=_F32)
    vq_ref[...] = 1.25 * ((q - st) ** 2).mean(axis=0, keepdims=True).mean(
        axis=1, keepdims=True)


def _k2(e1, fc12_w, fc12_b, fc21_w, fc21_b, bn_w, bn_b, codebook):
    return pl.pallas_call(
        _k2_body,
        grid=(1,),
        in_specs=[
            pl.BlockSpec((B, EU), lambda i: (0, 0)),
            pl.BlockSpec((EU, EU), lambda i: (0, 0)),
            pl.BlockSpec((1, EU), lambda i: (0, 0)),
            pl.BlockSpec((T, EU), lambda i: (0, 0)),
            pl.BlockSpec((1, T), lambda i: (0, 0)),
            pl.BlockSpec((1, T), lambda i: (0, 0)),
            pl.BlockSpec((1, T), lambda i: (0, 0)),
            pl.BlockSpec((T, T), lambda i: (0, 0)),
        ],
        out_specs=[
            pl.BlockSpec((B, T), lambda i: (0, 0)),
            pl.BlockSpec((B, T), lambda i: (0, 0)),
            pl.BlockSpec((1, T), lambda i: (0, 0)),
            pl.BlockSpec((1, 1), lambda i: (0, 0)),
        ],
        out_shape=[
            jax.ShapeDtypeStruct((B, T), _F32),
            jax.ShapeDtypeStruct((B, T), _F32),
            jax.ShapeDtypeStruct((1, T), _F32),
            jax.ShapeDtypeStruct((1, 1), _F32),
        ],
    )(e1, fc12_w, fc12_b, fc21_w, fc21_b, bn_w, bn_b, codebook)


def _k3_body(cb_ref, fcd1_ref, cnt_ref, dw_ref, db_ref, zn_ref, lse_ref):
    zcb = lax.dot_general(cb_ref[...], fcd1_ref[...], (((1,), (1,)), ((), ())),
                          preferred_element_type=_F32)
    w = cnt_ref[...] * (1.0 / B)
    zmu = lax.dot_general(w, zcb, (((1,), (0,)), ((), ())),
                          preferred_element_type=_F32)
    zc = zcb - zmu
    zvar = lax.dot_general(w, zc * zc, (((1,), (0,)), ((), ())),
                           preferred_element_type=_F32)
    zn = zc / jnp.sqrt(zvar + EPS) * dw_ref[...] + db_ref[...]
    zn_ref[...] = zn
    zm = zn.max(axis=1, keepdims=True)
    lse_ref[...] = zm + jnp.log(jnp.exp(zn - zm).sum(axis=1, keepdims=True))


def _k3(codebook, fcd1_w, counts, dec_bn_w, dec_bn_b):
    return pl.pallas_call(
        _k3_body,
        grid=(1,),
        in_specs=[
            pl.BlockSpec((T, T), lambda i: (0, 0)),
            pl.BlockSpec((V, T), lambda i: (0, 0)),
            pl.BlockSpec((1, T), lambda i: (0, 0)),
            pl.BlockSpec((1, V), lambda i: (0, 0)),
            pl.BlockSpec((1, V), lambda i: (0, 0)),
        ],
        out_specs=[
            pl.BlockSpec((T, V), lambda i: (0, 0)),
            pl.BlockSpec((T, 1), lambda i: (0, 0)),
        ],
        out_shape=[
            jax.ShapeDtypeStruct((T, V), _F32),
            jax.ShapeDtypeStruct((T, 1), _F32),
        ],
    )(codebook, fcd1_w, counts, dec_bn_w, dec_bn_b)


def _k4_body(td_ref, zn_ref, g_ref, rs_ref):
    td = td_ref[...]
    g_ref[...] = lax.dot_general(td, zn_ref[...], (((1,), (1,)), ((), ())),
                                 preferred_element_type=_F32)
    rs_ref[...] = td.sum(axis=1, keepdims=True)


def _k4(training_data, zn):
    bm = 256
    return pl.pallas_call(
        _k4_body,
        grid=(N // bm,),
        in_specs=[
            pl.BlockSpec((bm, V), lambda i: (i, 0)),
            pl.BlockSpec((T, V), lambda i: (0, 0)),
        ],
        out_specs=[
            pl.BlockSpec((bm, T), lambda i: (i, 0)),
            pl.BlockSpec((bm, 1), lambda i: (i, 0)),
        ],
        out_shape=[
            jax.ShapeDtypeStruct((N, T), _F32),
            jax.ShapeDtypeStruct((N, 1), _F32),
        ],
    )(training_data, zn)


def _k5_body(st_ref, oh_ref, xsum_ref, idxf_ref, cosg_ref, coog_ref,
             tb_ref, g_ref, rs_ref, lse_ref, t1_ref, t2_ref, t3_ref):
    i = pl.program_id(0)
    st = st_ref[...]
    tb = tb_ref[...]
    s2 = (st * st).sum(axis=1, keepdims=True)
    ones_row = jnp.ones((1, T), _F32)
    tb2 = lax.dot_general(ones_row, tb * tb, (((1,), (1,)), ((), ())),
                          preferred_element_type=_F32)
    cost = (s2 + tb2
            - 2.0 * lax.dot_general(st, tb, (((1,), (1,)), ((), ())),
                                    preferred_element_type=_F32))
    bow = RHO * cosg_ref[...] + (1.0 - RHO) * coog_ref[...]
    fuse = ETA * cost * cost + (1.0 - ETA) * bow
    ci = lax.broadcasted_iota(jnp.int32, fuse.shape, 1)
    fuse = jnp.where(ci == idxf_ref[...], jnp.inf, fuse)
    c_acc = jnp.zeros_like(fuse)
    for _ in range(K):
        m = fuse.min(axis=1, keepdims=True)
        a = jnp.where(fuse == m, ci, N).min(axis=1, keepdims=True)
        mask = ci == a
        c_acc = c_acc + mask.astype(_F32)
        fuse = jnp.where(mask, jnp.inf, fuse)
    r = lax.dot_general(c_acc, rs_ref[...], (((1,), (0,)), ((), ())),
                        preferred_element_type=_F32)
    cg = lax.dot_general(c_acc, g_ref[...], (((1,), (0,)), ((), ())),
                         preferred_element_type=_F32)
    lse_e = lax.dot_general(oh_ref[...], lse_ref[...], (((1,), (0,)), ((), ())),
                            preferred_element_type=_F32)
    def _sum11(x):
        return x.sum(axis=0, keepdims=True).sum(axis=1, keepdims=True)

    t1p = _sum11(cg * oh_ref[...])
    t2p = _sum11(xsum_ref[...] * lse_e)
    t3p = _sum11(r * lse_e)

    @pl.when(i == 0)
    def _init():
        t1_ref[...] = jnp.zeros_like(t1_ref)
        t2_ref[...] = jnp.zeros_like(t2_ref)
        t3_ref[...] = jnp.zeros_like(t3_ref)

    t1_ref[...] += t1p
    t2_ref[...] += t2p
    t3_ref[...] += t3p


def _k5(st, onehot, xsum, idx_f, cosg, coog, theta_bank, g, rs, lse):
    bm = 128
    return pl.pallas_call(
        _k5_body,
        grid=(B // bm,),
        in_specs=[
            pl.BlockSpec((bm, T), lambda i: (i, 0)),
            pl.BlockSpec((bm, T), lambda i: (i, 0)),
            pl.BlockSpec((bm, 1), lambda i: (i, 0)),
            pl.BlockSpec((bm, 1), lambda i: (i, 0)),
            pl.BlockSpec((bm, N), lambda i: (i, 0)),
            pl.BlockSpec((bm, N), lambda i: (i, 0)),
            pl.BlockSpec((N, T), lambda i: (0, 0)),
            pl.BlockSpec((N, T), lambda i: (0, 0)),
            pl.BlockSpec((N, 1), lambda i: (0, 0)),
            pl.BlockSpec((T, 1), lambda i: (0, 0)),
        ],
        out_specs=[
            pl.BlockSpec((1, 1), lambda i: (0, 0)),
            pl.BlockSpec((1, 1), lambda i: (0, 0)),
            pl.BlockSpec((1, 1), lambda i: (0, 0)),
        ],
        out_shape=[
            jax.ShapeDtypeStruct((1, 1), _F32),
            jax.ShapeDtypeStruct((1, 1), _F32),
            jax.ShapeDtypeStruct((1, 1), _F32),
        ],
    )(st, onehot, xsum, idx_f, cosg, coog, theta_bank, g, rs, lse)


def _k6_body(x_ref, oh_ref, zn_ref, t4_ref):
    i = pl.program_id(0)
    s = lax.dot_general(oh_ref[...], x_ref[...], (((0,), (0,)), ((), ())),
                        preferred_element_type=_F32)
    part = (s * zn_ref[...]).sum(axis=0, keepdims=True).sum(
        axis=1, keepdims=True)

    @pl.when(i == 0)
    def _init():
        t4_ref[...] = jnp.zeros_like(t4_ref)

    t4_ref[...] += part


def _k6(inputs, onehot, zn):
    bm = 256
    return pl.pallas_call(
        _k6_body,
        grid=(B // bm,),
        in_specs=[
            pl.BlockSpec((bm, V), lambda i: (i, 0)),
            pl.BlockSpec((bm, T), lambda i: (i, 0)),
            pl.BlockSpec((T, V), lambda i: (0, 0)),
        ],
        out_specs=[pl.BlockSpec((1, 1), lambda i: (0, 0))],
        out_shape=[jax.ShapeDtypeStruct((1, 1), _F32)],
    )(inputs, onehot, zn)


def kernel(inputs, idx, fc11_w, fc11_b, fc12_w, fc12_b, fc21_w, fc21_b,
           mean_bn_w, mean_bn_b, fcd1_w, dec_bn_w, dec_bn_b, codebook,
           theta_bank, M_cos_dist, M_coo_dist, training_data, is_aug):
    idx = idx.astype(jnp.int32)
    idx_col = idx.reshape(B, 1)

    cosg, coog = _bow_gather(idx, M_cos_dist, M_coo_dist)

    e1, xsum = _k1(inputs, fc11_w, fc11_b.reshape(1, EU))
    st, onehot, counts, vq = _k2(e1, fc12_w, fc12_b.reshape(1, EU), fc21_w,
                                 fc21_b.reshape(1, T), mean_bn_w.reshape(1, T),
                                 mean_bn_b.reshape(1, T), codebook)
    zn, lse = _k3(codebook, fcd1_w, counts, dec_bn_w.reshape(1, V),
                  dec_bn_b.reshape(1, V))
    g, rs = _k4(training_data, zn)
    t4 = _k6(inputs, onehot, zn)[0]
    t1, t2, t3 = _k5(st, onehot, xsum, idx_col, cosg, coog, theta_bank, g, rs,
                     lse)

    aug = jnp.where(is_aug, jnp.float32(1.0), jnp.float32(0.0))
    scale = ALPHA / K
    rec_loss = (1.0 / B) * (-(t4[0, 0] + aug * scale * t1[0, 0])
                            + t2[0, 0] + aug * scale * t3[0, 0])
    return rec_loss + vq[0, 0]


# speedup vs baseline: 7.1322x; 1.0707x over previous
import functools

import jax
import jax.numpy as jnp
from jax import lax
from jax.experimental import pallas as pl
from jax.experimental.pallas import tpu as pltpu
from jax.experimental.pallas import tpu_sc as plsc

B = 1024
V = 10000
N = 4096
T = 50
EU = 200
K = 20
ALPHA = 1.0
ETA = 0.5
RHO = 0.5
EPS = 1e-5

_F32 = jnp.float32


_CH = 8
_N_CH = 4

_sc_mesh = plsc.VectorSubcoreMesh(core_axis_name="c", subcore_axis_name="s")


@functools.partial(
    pl.kernel,
    mesh=_sc_mesh,
    out_type=(
        jax.ShapeDtypeStruct((B, N), _F32),
        jax.ShapeDtypeStruct((B, N), _F32),
    ),
    scratch_types=[
        pltpu.VMEM((_CH,), jnp.int32),
        pltpu.VMEM((_CH, N), _F32),
        pltpu.VMEM((_CH, N), _F32),
        pltpu.SemaphoreType.DMA,
        pltpu.SemaphoreType.DMA,
    ],
)
def _bow_gather(idx_hbm, cos_hbm, coo_hbm, cosg_hbm, coog_hbm,
                idx_v, a_v, b_v, sem_a, sem_b):
    wid = lax.axis_index("s") * 2 + lax.axis_index("c")
    base = wid * (B // 32)

    def chunk(c, carry):
        off = pl.multiple_of(base + c * _CH, _CH)
        pltpu.sync_copy(idx_hbm.at[pl.ds(off, _CH)], idx_v)
        cp_a = pltpu.async_copy(cos_hbm.at[idx_v], a_v, sem_a)
        cp_b = pltpu.async_copy(coo_hbm.at[idx_v], b_v, sem_b)
        cp_a.wait()
        cp_b.wait()
        pltpu.sync_copy(a_v, cosg_hbm.at[pl.ds(off, _CH)])
        pltpu.sync_copy(b_v, coog_hbm.at[pl.ds(off, _CH)])
        return carry

    lax.fori_loop(0, _N_CH, chunk, 0)


def _k1_body(x_ref, w_ref, b_ref, e1_ref, xsum_ref):
    x = x_ref[...]
    acc = lax.dot_general(x, w_ref[...], (((1,), (1,)), ((), ())),
                          preferred_element_type=_F32)
    e1_ref[...] = jnp.logaddexp(acc + b_ref[...], 0.0)
    xsum_ref[...] = x.sum(axis=1, keepdims=True)


def _k1(inputs, fc11_w, fc11_b):
    bm = 256
    return pl.pallas_call(
        _k1_body,
        grid=(B // bm,),
        in_specs=[
            pl.BlockSpec((bm, V), lambda i: (i, 0)),
            pl.BlockSpec((EU, V), lambda i: (0, 0)),
            pl.BlockSpec((1, EU), lambda i: (0, 0)),
        ],
        out_specs=[
            pl.BlockSpec((bm, EU), lambda i: (i, 0)),
            pl.BlockSpec((bm, 1), lambda i: (i, 0)),
        ],
        out_shape=[
            jax.ShapeDtypeStruct((B, EU), _F32),
            jax.ShapeDtypeStruct((B, 1), _F32),
        ],
    )(inputs, fc11_w, fc11_b)


def _k2_body(e1_ref, w12_ref, b12_ref, w21_ref, b21_ref, bnw_ref, bnb_ref,
             cb_ref, st_ref, oh_ref, cnt_ref, vq_ref):
    e1 = e1_ref[...]
    e2 = jnp.logaddexp(
        lax.dot_general(e1, w12_ref[...], (((1,), (1,)), ((), ())),
                        preferred_element_type=_F32) + b12_ref[...], 0.0)
    h = lax.dot_general(e2, w21_ref[...], (((1,), (1,)), ((), ())),
                        preferred_element_type=_F32) + b21_ref[...]
    mu = jnp.mean(h, axis=0, keepdims=True)
    var = jnp.mean((h - mu) ** 2, axis=0, keepdims=True)
    theta = (h - mu) / jnp.sqrt(var + EPS) * bnw_ref[...] + bnb_ref[...]
    tmax = theta.max(axis=1, keepdims=True)
    ex = jnp.exp(theta - tmax)
    st = ex / ex.sum(axis=1, keepdims=True)
    st_ref[...] = st
    cb = cb_ref[...]
    d = ((st * st).sum(axis=1, keepdims=True)
         + (cb * cb).sum(axis=1, keepdims=True).reshape(1, T)
         - 2.0 * lax.dot_general(st, cb, (((1,), (1,)), ((), ())),
                                 preferred_element_type=_F32))
    dmin = d.min(axis=1, keepdims=True)
    ii = lax.broadcasted_iota(jnp.int32, d.shape, 1)
    a = jnp.where(d == dmin, ii, T).min(axis=1, keepdims=True)
    oh = (ii == a).astype(_F32)
    oh_ref[...] = oh
    cnt_ref[...] = oh.sum(axis=0, keepdims=True)
    q = lax.dot_general(oh, cb, (((1,), (0,)), ((), ())),
                        preferred_element_type=_F32)
    vq_ref[...] = 1.25 * ((q - st) ** 2).mean(axis=0, keepdims=True).mean(
        axis=1, keepdims=True)


def _k2(e1, fc12_w, fc12_b, fc21_w, fc21_b, bn_w, bn_b, codebook):
    return pl.pallas_call(
        _k2_body,
        grid=(1,),
        in_specs=[
            pl.BlockSpec((B, EU), lambda i: (0, 0)),
            pl.BlockSpec((EU, EU), lambda i: (0, 0)),
            pl.BlockSpec((1, EU), lambda i: (0, 0)),
            pl.BlockSpec((T, EU), lambda i: (0, 0)),
            pl.BlockSpec((1, T), lambda i: (0, 0)),
            pl.BlockSpec((1, T), lambda i: (0, 0)),
            pl.BlockSpec((1, T), lambda i: (0, 0)),
            pl.BlockSpec((T, T), lambda i: (0, 0)),
        ],
        out_specs=[
            pl.BlockSpec((B, T), lambda i: (0, 0)),
            pl.BlockSpec((B, T), lambda i: (0, 0)),
            pl.BlockSpec((1, T), lambda i: (0, 0)),
            pl.BlockSpec((1, 1), lambda i: (0, 0)),
        ],
        out_shape=[
            jax.ShapeDtypeStruct((B, T), _F32),
            jax.ShapeDtypeStruct((B, T), _F32),
            jax.ShapeDtypeStruct((1, T), _F32),
            jax.ShapeDtypeStruct((1, 1), _F32),
        ],
    )(e1, fc12_w, fc12_b, fc21_w, fc21_b, bn_w, bn_b, codebook)


def _k3_body(cb_ref, fcd1_ref, cnt_ref, dw_ref, db_ref, zn_ref, lse_ref):
    zcb = lax.dot_general(cb_ref[...], fcd1_ref[...], (((1,), (1,)), ((), ())),
                          preferred_element_type=_F32)
    w = cnt_ref[...] * (1.0 / B)
    zmu = lax.dot_general(w, zcb, (((1,), (0,)), ((), ())),
                          preferred_element_type=_F32)
    zc = zcb - zmu
    zvar = lax.dot_general(w, zc * zc, (((1,), (0,)), ((), ())),
                           preferred_element_type=_F32)
    zn = zc / jnp.sqrt(zvar + EPS) * dw_ref[...] + db_ref[...]
    zn_ref[...] = zn
    zm = zn.max(axis=1, keepdims=True)
    lse_ref[...] = zm + jnp.log(jnp.exp(zn - zm).sum(axis=1, keepdims=True))


def _k3(codebook, fcd1_w, counts, dec_bn_w, dec_bn_b):
    return pl.pallas_call(
        _k3_body,
        grid=(1,),
        in_specs=[
            pl.BlockSpec((T, T), lambda i: (0, 0)),
            pl.BlockSpec((V, T), lambda i: (0, 0)),
            pl.BlockSpec((1, T), lambda i: (0, 0)),
            pl.BlockSpec((1, V), lambda i: (0, 0)),
            pl.BlockSpec((1, V), lambda i: (0, 0)),
        ],
        out_specs=[
            pl.BlockSpec((T, V), lambda i: (0, 0)),
            pl.BlockSpec((T, 1), lambda i: (0, 0)),
        ],
        out_shape=[
            jax.ShapeDtypeStruct((T, V), _F32),
            jax.ShapeDtypeStruct((T, 1), _F32),
        ],
    )(codebook, fcd1_w, counts, dec_bn_w, dec_bn_b)


def _k4_body(td_ref, zn_ref, g_ref, rs_ref):
    td = td_ref[...]
    g_ref[...] = lax.dot_general(td, zn_ref[...], (((1,), (1,)), ((), ())),
                                 preferred_element_type=_F32)
    rs_ref[...] = td.sum(axis=1, keepdims=True)


def _k4(training_data, zn):
    bm = 256
    return pl.pallas_call(
        _k4_body,
        grid=(N // bm,),
        in_specs=[
            pl.BlockSpec((bm, V), lambda i: (i, 0)),
            pl.BlockSpec((T, V), lambda i: (0, 0)),
        ],
        out_specs=[
            pl.BlockSpec((bm, T), lambda i: (i, 0)),
            pl.BlockSpec((bm, 1), lambda i: (i, 0)),
        ],
        out_shape=[
            jax.ShapeDtypeStruct((N, T), _F32),
            jax.ShapeDtypeStruct((N, 1), _F32),
        ],
    )(training_data, zn)


def _k5_body(st_ref, oh_ref, xsum_ref, idxf_ref, cosg_ref, coog_ref,
             tb_ref, g_ref, rs_ref, lse_ref, t1_ref, t2_ref, t3_ref):
    i = pl.program_id(0)
    st = st_ref[...]
    tb = tb_ref[...]
    s2 = (st * st).sum(axis=1, keepdims=True)
    ones_row = jnp.ones((1, T), _F32)
    tb2 = lax.dot_general(ones_row, tb * tb, (((1,), (1,)), ((), ())),
                          preferred_element_type=_F32)
    cost = (s2 + tb2
            - 2.0 * lax.dot_general(st, tb, (((1,), (1,)), ((), ())),
                                    preferred_element_type=_F32))
    bow = RHO * cosg_ref[...] + (1.0 - RHO) * coog_ref[...]
    fuse = ETA * cost * cost + (1.0 - ETA) * bow
    ci = lax.broadcasted_iota(jnp.int32, fuse.shape, 1)
    key = jnp.where(ci == idxf_ref[...], jnp.int32(0x7F800000),
                    lax.bitcast_convert_type(fuse, jnp.int32))
    key = (key & jnp.int32(~0xFFF)) | ci
    c_acc = jnp.zeros(fuse.shape, _F32)
    big = jnp.int32(0x7FFFF000)
    for _ in range(K):
        kmin = key.min(axis=1, keepdims=True)
        mask = key == kmin
        c_acc = c_acc + jnp.where(mask, _F32(1.0), _F32(0.0))
        key = jnp.where(mask, big, key)
    r = lax.dot_general(c_acc, rs_ref[...], (((1,), (0,)), ((), ())),
                        preferred_element_type=_F32)
    cg = lax.dot_general(c_acc, g_ref[...], (((1,), (0,)), ((), ())),
                         preferred_element_type=_F32)
    lse_e = lax.dot_general(oh_ref[...], lse_ref[...], (((1,), (0,)), ((), ())),
                            preferred_element_type=_F32)
    def _sum11(x):
        return x.sum(axis=0, keepdims=True).sum(axis=1, keepdims=True)

    t1p = _sum11(cg * oh_ref[...])
    t2p = _sum11(xsum_ref[...] * lse_e)
    t3p = _sum11(r * lse_e)

    @pl.when(i == 0)
    def _init():
        t1_ref[...] = jnp.zeros_like(t1_ref)
        t2_ref[...] = jnp.zeros_like(t2_ref)
        t3_ref[...] = jnp.zeros_like(t3_ref)

    t1_ref[...] += t1p
    t2_ref[...] += t2p
    t3_ref[...] += t3p


def _k5(st, onehot, xsum, idx_f, cosg, coog, theta_bank, g, rs, lse):
    bm = 128
    return pl.pallas_call(
        _k5_body,
        grid=(B // bm,),
        in_specs=[
            pl.BlockSpec((bm, T), lambda i: (i, 0)),
            pl.BlockSpec((bm, T), lambda i: (i, 0)),
            pl.BlockSpec((bm, 1), lambda i: (i, 0)),
            pl.BlockSpec((bm, 1), lambda i: (i, 0)),
            pl.BlockSpec((bm, N), lambda i: (i, 0)),
            pl.BlockSpec((bm, N), lambda i: (i, 0)),
            pl.BlockSpec((N, T), lambda i: (0, 0)),
            pl.BlockSpec((N, T), lambda i: (0, 0)),
            pl.BlockSpec((N, 1), lambda i: (0, 0)),
            pl.BlockSpec((T, 1), lambda i: (0, 0)),
        ],
        out_specs=[
            pl.BlockSpec((1, 1), lambda i: (0, 0)),
            pl.BlockSpec((1, 1), lambda i: (0, 0)),
            pl.BlockSpec((1, 1), lambda i: (0, 0)),
        ],
        out_shape=[
            jax.ShapeDtypeStruct((1, 1), _F32),
            jax.ShapeDtypeStruct((1, 1), _F32),
            jax.ShapeDtypeStruct((1, 1), _F32),
        ],
    )(st, onehot, xsum, idx_f, cosg, coog, theta_bank, g, rs, lse)


def _k6_body(x_ref, oh_ref, zn_ref, t4_ref):
    i = pl.program_id(0)
    s = lax.dot_general(oh_ref[...], x_ref[...], (((0,), (0,)), ((), ())),
                        preferred_element_type=_F32)
    part = (s * zn_ref[...]).sum(axis=0, keepdims=True).sum(
        axis=1, keepdims=True)

    @pl.when(i == 0)
    def _init():
        t4_ref[...] = jnp.zeros_like(t4_ref)

    t4_ref[...] += part


def _k6(inputs, onehot, zn):
    bm = 256
    return pl.pallas_call(
        _k6_body,
        grid=(B // bm,),
        in_specs=[
            pl.BlockSpec((bm, V), lambda i: (i, 0)),
            pl.BlockSpec((bm, T), lambda i: (i, 0)),
            pl.BlockSpec((T, V), lambda i: (0, 0)),
        ],
        out_specs=[pl.BlockSpec((1, 1), lambda i: (0, 0))],
        out_shape=[jax.ShapeDtypeStruct((1, 1), _F32)],
    )(inputs, onehot, zn)


def kernel(inputs, idx, fc11_w, fc11_b, fc12_w, fc12_b, fc21_w, fc21_b,
           mean_bn_w, mean_bn_b, fcd1_w, dec_bn_w, dec_bn_b, codebook,
           theta_bank, M_cos_dist, M_coo_dist, training_data, is_aug):
    idx = idx.astype(jnp.int32)
    idx_col = idx.reshape(B, 1)

    cosg, coog = _bow_gather(idx, M_cos_dist, M_coo_dist)

    e1, xsum = _k1(inputs, fc11_w, fc11_b.reshape(1, EU))
    st, onehot, counts, vq = _k2(e1, fc12_w, fc12_b.reshape(1, EU), fc21_w,
                                 fc21_b.reshape(1, T), mean_bn_w.reshape(1, T),
                                 mean_bn_b.reshape(1, T), codebook)
    zn, lse = _k3(codebook, fcd1_w, counts, dec_bn_w.reshape(1, V),
                  dec_bn_b.reshape(1, V))
    g, rs = _k4(training_data, zn)
    t4 = _k6(inputs, onehot, zn)[0]
    t1, t2, t3 = _k5(st, onehot, xsum, idx_col, cosg, coog, theta_bank, g, rs,
                     lse)

    aug = jnp.where(is_aug, jnp.float32(1.0), jnp.float32(0.0))
    scale = ALPHA / K
    rec_loss = (1.0 / B) * (-(t4[0, 0] + aug * scale * t1[0, 0])
                            + t2[0, 0] + aug * scale * t3[0, 0])
    return rec_loss + vq[0, 0]


# speedup vs baseline: 7.1953x; 1.0089x over previous
import functools

import jax
import jax.numpy as jnp
from jax import lax
from jax.experimental import pallas as pl
from jax.experimental.pallas import tpu as pltpu
from jax.experimental.pallas import tpu_sc as plsc

B = 1024
V = 10000
N = 4096
T = 50
EU = 200
K = 20
ALPHA = 1.0
ETA = 0.5
RHO = 0.5
EPS = 1e-5

_F32 = jnp.float32


_CH = 8
_N_CH = 4

_sc_mesh = plsc.VectorSubcoreMesh(core_axis_name="c", subcore_axis_name="s")


@functools.partial(
    pl.kernel,
    mesh=_sc_mesh,
    out_type=jax.ShapeDtypeStruct((B, N), _F32),
    scratch_types=[
        pltpu.VMEM((_CH,), jnp.int32),
        pltpu.VMEM((_CH, N), _F32),
        pltpu.VMEM((_CH, N), _F32),
        pltpu.SemaphoreType.DMA,
        pltpu.SemaphoreType.DMA,
    ],
)
def _bow_gather(idx_hbm, cos_hbm, coo_hbm, bow_hbm,
                idx_v, a_v, b_v, sem_a, sem_b):
    wid = lax.axis_index("s") * 2 + lax.axis_index("c")
    base = wid * (B // 32)

    def chunk(c, carry):
        off = pl.multiple_of(base + c * _CH, _CH)
        pltpu.sync_copy(idx_hbm.at[pl.ds(off, _CH)], idx_v)
        cp_a = pltpu.async_copy(cos_hbm.at[idx_v], a_v, sem_a)
        cp_b = pltpu.async_copy(coo_hbm.at[idx_v], b_v, sem_b)
        cp_a.wait()
        cp_b.wait()

        def blend(j, carry2):
            sl = pl.ds(j * 16, 16)
            for r in range(_CH):
                a_v[r, sl] = RHO * a_v[r, sl] + (1.0 - RHO) * b_v[r, sl]
            return carry2

        lax.fori_loop(0, N // 16, blend, 0)
        pltpu.sync_copy(a_v, bow_hbm.at[pl.ds(off, _CH)])
        return carry

    lax.fori_loop(0, _N_CH, chunk, 0)


def _k1_body(x_ref, w_ref, b_ref, e1_ref, xsum_ref):
    x = x_ref[...]
    acc = lax.dot_general(x, w_ref[...], (((1,), (1,)), ((), ())),
                          preferred_element_type=_F32)
    e1_ref[...] = jnp.logaddexp(acc + b_ref[...], 0.0)
    xsum_ref[...] = x.sum(axis=1, keepdims=True)


def _k1(inputs, fc11_w, fc11_b):
    bm = 256
    return pl.pallas_call(
        _k1_body,
        grid=(B // bm,),
        in_specs=[
            pl.BlockSpec((bm, V), lambda i: (i, 0)),
            pl.BlockSpec((EU, V), lambda i: (0, 0)),
            pl.BlockSpec((1, EU), lambda i: (0, 0)),
        ],
        out_specs=[
            pl.BlockSpec((bm, EU), lambda i: (i, 0)),
            pl.BlockSpec((bm, 1), lambda i: (i, 0)),
        ],
        out_shape=[
            jax.ShapeDtypeStruct((B, EU), _F32),
            jax.ShapeDtypeStruct((B, 1), _F32),
        ],
    )(inputs, fc11_w, fc11_b)


def _k2_body(e1_ref, w12_ref, b12_ref, w21_ref, b21_ref, bnw_ref, bnb_ref,
             cb_ref, fcd1_ref, dw_ref, db_ref,
             st_ref, oh_ref, vq_ref, zn_ref, lse_ref):
    e1 = e1_ref[...]
    e2 = jnp.logaddexp(
        lax.dot_general(e1, w12_ref[...], (((1,), (1,)), ((), ())),
                        preferred_element_type=_F32) + b12_ref[...], 0.0)
    h = lax.dot_general(e2, w21_ref[...], (((1,), (1,)), ((), ())),
                        preferred_element_type=_F32) + b21_ref[...]
    mu = jnp.mean(h, axis=0, keepdims=True)
    var = jnp.mean((h - mu) ** 2, axis=0, keepdims=True)
    theta = (h - mu) / jnp.sqrt(var + EPS) * bnw_ref[...] + bnb_ref[...]
    tmax = theta.max(axis=1, keepdims=True)
    ex = jnp.exp(theta - tmax)
    st = ex / ex.sum(axis=1, keepdims=True)
    st_ref[...] = st
    cb = cb_ref[...]
    d = ((st * st).sum(axis=1, keepdims=True)
         + (cb * cb).sum(axis=1, keepdims=True).reshape(1, T)
         - 2.0 * lax.dot_general(st, cb, (((1,), (1,)), ((), ())),
                                 preferred_element_type=_F32))
    dmin = d.min(axis=1, keepdims=True)
    ii = lax.broadcasted_iota(jnp.int32, d.shape, 1)
    a = jnp.where(d == dmin, ii, T).min(axis=1, keepdims=True)
    oh = (ii == a).astype(_F32)
    oh_ref[...] = oh
    cnt = oh.sum(axis=0, keepdims=True)
    q = lax.dot_general(oh, cb, (((1,), (0,)), ((), ())),
                        preferred_element_type=_F32)
    vq_ref[...] = 1.25 * ((q - st) ** 2).mean(axis=0, keepdims=True).mean(
        axis=1, keepdims=True)
    zcb = lax.dot_general(cb, fcd1_ref[...], (((1,), (1,)), ((), ())),
                          preferred_element_type=_F32)
    w = cnt * (1.0 / B)
    zmu = lax.dot_general(w, zcb, (((1,), (0,)), ((), ())),
                          preferred_element_type=_F32)
    zc = zcb - zmu
    zvar = lax.dot_general(w, zc * zc, (((1,), (0,)), ((), ())),
                           preferred_element_type=_F32)
    zn = zc / jnp.sqrt(zvar + EPS) * dw_ref[...] + db_ref[...]
    zn_ref[...] = zn
    zm = zn.max(axis=1, keepdims=True)
    lse_ref[...] = zm + jnp.log(jnp.exp(zn - zm).sum(axis=1, keepdims=True))


def _k2(e1, fc12_w, fc12_b, fc21_w, fc21_b, bn_w, bn_b, codebook, fcd1_w,
        dec_bn_w, dec_bn_b):
    return pl.pallas_call(
        _k2_body,
        grid=(1,),
        in_specs=[
            pl.BlockSpec((B, EU), lambda i: (0, 0)),
            pl.BlockSpec((EU, EU), lambda i: (0, 0)),
            pl.BlockSpec((1, EU), lambda i: (0, 0)),
            pl.BlockSpec((T, EU), lambda i: (0, 0)),
            pl.BlockSpec((1, T), lambda i: (0, 0)),
            pl.BlockSpec((1, T), lambda i: (0, 0)),
            pl.BlockSpec((1, T), lambda i: (0, 0)),
            pl.BlockSpec((T, T), lambda i: (0, 0)),
            pl.BlockSpec((V, T), lambda i: (0, 0)),
            pl.BlockSpec((1, V), lambda i: (0, 0)),
            pl.BlockSpec((1, V), lambda i: (0, 0)),
        ],
        out_specs=[
            pl.BlockSpec((B, T), lambda i: (0, 0)),
            pl.BlockSpec((B, T), lambda i: (0, 0)),
            pl.BlockSpec((1, 1), lambda i: (0, 0)),
            pl.BlockSpec((T, V), lambda i: (0, 0)),
            pl.BlockSpec((T, 1), lambda i: (0, 0)),
        ],
        out_shape=[
            jax.ShapeDtypeStruct((B, T), _F32),
            jax.ShapeDtypeStruct((B, T), _F32),
            jax.ShapeDtypeStruct((1, 1), _F32),
            jax.ShapeDtypeStruct((T, V), _F32),
            jax.ShapeDtypeStruct((T, 1), _F32),
        ],
    )(e1, fc12_w, fc12_b, fc21_w, fc21_b, bn_w, bn_b, codebook, fcd1_w,
      dec_bn_w, dec_bn_b)


def _k4_body(td_ref, zn_ref, x_ref, oh_ref, g_ref, rs_ref, t4_ref):
    i = pl.program_id(0)
    td = td_ref[...]
    g_ref[...] = lax.dot_general(td, zn_ref[...], (((1,), (1,)), ((), ())),
                                 preferred_element_type=_F32)
    rs_ref[...] = td.sum(axis=1, keepdims=True)

    @pl.when(i == 0)
    def _init():
        t4_ref[...] = jnp.zeros_like(t4_ref)

    @pl.when(i % 4 == 0)
    def _accum():
        s = lax.dot_general(oh_ref[...], x_ref[...], (((0,), (0,)), ((), ())),
                            preferred_element_type=_F32)
        t4_ref[...] += (s * zn_ref[...]).sum(axis=0, keepdims=True).sum(
            axis=1, keepdims=True)


def _k4(training_data, zn, inputs, onehot):
    bm = 256
    return pl.pallas_call(
        _k4_body,
        grid=(N // bm,),
        in_specs=[
            pl.BlockSpec((bm, V), lambda i: (i, 0)),
            pl.BlockSpec((T, V), lambda i: (0, 0)),
            pl.BlockSpec((bm, V), lambda i: (i // 4, 0)),
            pl.BlockSpec((bm, T), lambda i: (i // 4, 0)),
        ],
        out_specs=[
            pl.BlockSpec((bm, T), lambda i: (i, 0)),
            pl.BlockSpec((bm, 1), lambda i: (i, 0)),
            pl.BlockSpec((1, 1), lambda i: (0, 0)),
        ],
        out_shape=[
            jax.ShapeDtypeStruct((N, T), _F32),
            jax.ShapeDtypeStruct((N, 1), _F32),
            jax.ShapeDtypeStruct((1, 1), _F32),
        ],
    )(training_data, zn, inputs, onehot)


def _k5_body(st_ref, oh_ref, xsum_ref, idxf_ref, bow_ref,
             tb_ref, g_ref, rs_ref, lse_ref, t1_ref, t2_ref, t3_ref):
    i = pl.program_id(0)
    st = st_ref[...]
    tb = tb_ref[...]
    s2 = (st * st).sum(axis=1, keepdims=True)
    ones_row = jnp.ones((1, T), _F32)
    tb2 = lax.dot_general(ones_row, tb * tb, (((1,), (1,)), ((), ())),
                          preferred_element_type=_F32)
    cost = (s2 + tb2
            - 2.0 * lax.dot_general(st, tb, (((1,), (1,)), ((), ())),
                                    preferred_element_type=_F32))
    fuse = ETA * cost * cost + (1.0 - ETA) * bow_ref[...]
    ci = lax.broadcasted_iota(jnp.int32, fuse.shape, 1)
    key = jnp.where(ci == idxf_ref[...], jnp.int32(0x7F800000),
                    lax.bitcast_convert_type(fuse, jnp.int32))
    key = (key & jnp.int32(~0xFFF)) | ci
    c_acc = jnp.zeros(fuse.shape, _F32)
    big = jnp.int32(0x7FFFF000)
    for _ in range(K):
        kmin = key.min(axis=1, keepdims=True)
        mask = key == kmin
        c_acc = c_acc + jnp.where(mask, _F32(1.0), _F32(0.0))
        key = jnp.where(mask, big, key)
    r = lax.dot_general(c_acc, rs_ref[...], (((1,), (0,)), ((), ())),
                        preferred_element_type=_F32)
    cg = lax.dot_general(c_acc, g_ref[...], (((1,), (0,)), ((), ())),
                         preferred_element_type=_F32)
    lse_e = lax.dot_general(oh_ref[...], lse_ref[...], (((1,), (0,)), ((), ())),
                            preferred_element_type=_F32)
    def _sum11(x):
        return x.sum(axis=0, keepdims=True).sum(axis=1, keepdims=True)

    t1p = _sum11(cg * oh_ref[...])
    t2p = _sum11(xsum_ref[...] * lse_e)
    t3p = _sum11(r * lse_e)

    @pl.when(i == 0)
    def _init():
        t1_ref[...] = jnp.zeros_like(t1_ref)
        t2_ref[...] = jnp.zeros_like(t2_ref)
        t3_ref[...] = jnp.zeros_like(t3_ref)

    t1_ref[...] += t1p
    t2_ref[...] += t2p
    t3_ref[...] += t3p


def _k5(st, onehot, xsum, idx_f, bow, theta_bank, g, rs, lse):
    bm = 128
    return pl.pallas_call(
        _k5_body,
        grid=(B // bm,),
        in_specs=[
            pl.BlockSpec((bm, T), lambda i: (i, 0)),
            pl.BlockSpec((bm, T), lambda i: (i, 0)),
            pl.BlockSpec((bm, 1), lambda i: (i, 0)),
            pl.BlockSpec((bm, 1), lambda i: (i, 0)),
            pl.BlockSpec((bm, N), lambda i: (i, 0)),
            pl.BlockSpec((N, T), lambda i: (0, 0)),
            pl.BlockSpec((N, T), lambda i: (0, 0)),
            pl.BlockSpec((N, 1), lambda i: (0, 0)),
            pl.BlockSpec((T, 1), lambda i: (0, 0)),
        ],
        out_specs=[
            pl.BlockSpec((1, 1), lambda i: (0, 0)),
            pl.BlockSpec((1, 1), lambda i: (0, 0)),
            pl.BlockSpec((1, 1), lambda i: (0, 0)),
        ],
        out_shape=[
            jax.ShapeDtypeStruct((1, 1), _F32),
            jax.ShapeDtypeStruct((1, 1), _F32),
            jax.ShapeDtypeStruct((1, 1), _F32),
        ],
    )(st, onehot, xsum, idx_f, bow, theta_bank, g, rs, lse)


def kernel(inputs, idx, fc11_w, fc11_b, fc12_w, fc12_b, fc21_w, fc21_b,
           mean_bn_w, mean_bn_b, fcd1_w, dec_bn_w, dec_bn_b, codebook,
           theta_bank, M_cos_dist, M_coo_dist, training_data, is_aug):
    idx = idx.astype(jnp.int32)
    idx_col = idx.reshape(B, 1)

    bow = _bow_gather(idx, M_cos_dist, M_coo_dist)

    e1, xsum = _k1(inputs, fc11_w, fc11_b.reshape(1, EU))
    st, onehot, vq, zn, lse = _k2(
        e1, fc12_w, fc12_b.reshape(1, EU), fc21_w, fc21_b.reshape(1, T),
        mean_bn_w.reshape(1, T), mean_bn_b.reshape(1, T), codebook, fcd1_w,
        dec_bn_w.reshape(1, V), dec_bn_b.reshape(1, V))
    g, rs, t4 = _k4(training_data, zn, inputs, onehot)
    t1, t2, t3 = _k5(st, onehot, xsum, idx_col, bow, theta_bank, g, rs, lse)

    aug = jnp.where(is_aug, jnp.float32(1.0), jnp.float32(0.0))
    scale = ALPHA / K
    rec_loss = (1.0 / B) * (-(t4[0, 0] + aug * scale * t1[0, 0])
                            + t2[0, 0] + aug * scale * t3[0, 0])
    return rec_loss + vq[0, 0]
